# baseline (speedup 1.0000x reference)
import functools

import jax
import jax.numpy as jnp
from jax import lax
from jax.experimental import pallas as pl
from jax.experimental.pallas import tpu as pltpu
from jax.experimental.pallas import tpu_sc as plsc

BATCH = 16384
HIST = 200
EMBED = 2
N_ROWS = 1_000_000
N_TOTAL = BATCH * HIST
NC, NS = 1, 16
NW = NC * NS
PER_W = N_TOTAL // NW
CHUNK = 12800
STEPS = PER_W // CHUNK
LANES = 16

_mesh = plsc.VectorSubcoreMesh(core_axis_name="c", subcore_axis_name="s", num_cores=NC, num_subcores=NS)


@functools.partial(
    pl.kernel,
    out_type=jax.ShapeDtypeStruct((N_TOTAL * EMBED,), jnp.float32),
    mesh=_mesh,
    scratch_types=[
        pltpu.VMEM((CHUNK,), jnp.int32),
        pltpu.VMEM((CHUNK * EMBED,), jnp.int32),
        pltpu.VMEM((CHUNK * EMBED,), jnp.float32),
        pltpu.SemaphoreType.DMA,
    ],
    compiler_params=pltpu.CompilerParams(
        use_tc_tiling_on_sc=False, needs_layout_passes=False),
)
def _gather_kernel(idx2_hbm, table_hbm, out_hbm, idx_v, didx_v, vals_v, sem):
    wid = lax.axis_index("s") * NC + lax.axis_index("c")
    base = wid * PER_W
    pos_e = lax.iota(jnp.int32, LANES) * 2
    pos_o = pos_e + 1

    def body(g, carry):
        off = base + g * CHUNK
        pltpu.sync_copy(idx2_hbm.at[pl.ds(off, CHUNK)], idx_v)

        def expand(k, c):
            v = idx_v[pl.ds(k * LANES, LANES)]
            p = pos_e + k * (2 * LANES)
            plsc.store_scatter(didx_v, [p], v)
            plsc.store_scatter(didx_v, [p + 1], v + 1)
            return c

        lax.fori_loop(0, CHUNK // LANES, expand, 0)

        pltpu.async_copy(table_hbm.at[didx_v], vals_v, sem).wait()
        pltpu.sync_copy(vals_v, out_hbm.at[pl.ds(off * EMBED, CHUNK * EMBED)])
        return carry

    lax.fori_loop(0, STEPS, body, 0)


def kernel(x, W):
    idx2 = x.reshape(N_TOTAL).astype(jnp.int32) * 2
    out = _gather_kernel(idx2, W.reshape(N_ROWS * EMBED))
    return out.reshape(BATCH, HIST, EMBED)


# speedup vs baseline: 1.0700x; 1.0700x over previous
import functools

import jax
import jax.numpy as jnp
from jax import lax
from jax.experimental import pallas as pl
from jax.experimental.pallas import tpu as pltpu
from jax.experimental.pallas import tpu_sc as plsc

BATCH = 16384
HIST = 200
EMBED = 2
N_ROWS = 1_000_000
N_TOTAL = BATCH * HIST
NC, NS = 2, 16
NW = NC * NS
PER_W = N_TOTAL // NW
CHUNK = 12800
STEPS = PER_W // CHUNK
LANES = 16

_mesh = plsc.VectorSubcoreMesh(core_axis_name="c", subcore_axis_name="s", num_cores=NC, num_subcores=NS)


@functools.partial(
    pl.kernel,
    out_type=jax.ShapeDtypeStruct((N_TOTAL * EMBED,), jnp.float32),
    mesh=_mesh,
    scratch_types=[
        pltpu.VMEM((CHUNK,), jnp.int32),
        pltpu.VMEM((CHUNK * EMBED,), jnp.int32),
        pltpu.VMEM((CHUNK * EMBED,), jnp.float32),
        pltpu.SemaphoreType.DMA,
    ],
    compiler_params=pltpu.CompilerParams(
        use_tc_tiling_on_sc=False, needs_layout_passes=False),
)
def _gather_kernel(idx2_hbm, table_hbm, out_hbm, idx_v, didx_v, vals_v, sem):
    wid = lax.axis_index("s") * NC + lax.axis_index("c")
    base = wid * PER_W
    pos_e = lax.iota(jnp.int32, LANES) * 2
    pos_o = pos_e + 1

    def body(g, carry):
        off = base + g * CHUNK
        pltpu.sync_copy(idx2_hbm.at[pl.ds(off, CHUNK)], idx_v)

        def expand(k, c):
            v = idx_v[pl.ds(k * LANES, LANES)]
            p = pos_e + k * (2 * LANES)
            plsc.store_scatter(didx_v, [p], v)
            plsc.store_scatter(didx_v, [p + 1], v + 1)
            return c

        lax.fori_loop(0, CHUNK // LANES, expand, 0)

        pltpu.async_copy(table_hbm.at[didx_v], vals_v, sem).wait()
        pltpu.sync_copy(vals_v, out_hbm.at[pl.ds(off * EMBED, CHUNK * EMBED)])
        return carry

    lax.fori_loop(0, STEPS, body, 0)


def kernel(x, W):
    idx2 = x.reshape(N_TOTAL).astype(jnp.int32) * 2
    wf = W.reshape(N_ROWS * EMBED)
    idx2, wf = lax.optimization_barrier((idx2, wf))
    out = lax.optimization_barrier(_gather_kernel(idx2, wf))
    return out.reshape(BATCH, HIST, EMBED)


# speedup vs baseline: 9.9676x; 9.3158x over previous
import functools

import jax
import jax.numpy as jnp
from jax import lax
from jax.experimental import pallas as pl
from jax.experimental.pallas import tpu as pltpu
from jax.experimental.pallas import tpu_sc as plsc

BATCH = 16384
HIST = 200
EMBED = 2
N_ROWS = 1_000_000
N_TOTAL = BATCH * HIST
NC, NS = 2, 16
NW = NC * NS
PER_W = N_TOTAL // NW
CHUNK = 12800
STEPS = PER_W // CHUNK

_mesh = plsc.VectorSubcoreMesh(core_axis_name="c", subcore_axis_name="s",
                               num_cores=NC, num_subcores=NS)


@functools.partial(
    pl.kernel,
    out_type=jax.ShapeDtypeStruct((EMBED * N_TOTAL,), jnp.float32),
    mesh=_mesh,
    scratch_types=[
        pltpu.VMEM((CHUNK,), jnp.int32),
        pltpu.VMEM((CHUNK,), jnp.float32),
        pltpu.VMEM((CHUNK,), jnp.float32),
        pltpu.SemaphoreType.DMA,
        pltpu.SemaphoreType.DMA,
    ],
    compiler_params=pltpu.CompilerParams(
        use_tc_tiling_on_sc=False, needs_layout_passes=False),
)
def _gather_kernel(idx_hbm, w0_hbm, w1_hbm, out_hbm, idx_v, e_v, o_v,
                   sem0, sem1):
    wid = lax.axis_index("s") * NC + lax.axis_index("c")
    base = wid * PER_W

    def body(g, carry):
        off = base + g * CHUNK
        pltpu.sync_copy(idx_hbm.at[pl.ds(off, CHUNK)], idx_v)
        c0 = pltpu.async_copy(w0_hbm.at[idx_v], e_v, sem0)
        c1 = pltpu.async_copy(w1_hbm.at[idx_v], o_v, sem1)
        c0.wait()
        pltpu.sync_copy(e_v, out_hbm.at[pl.ds(off, CHUNK)])
        c1.wait()
        pltpu.sync_copy(o_v, out_hbm.at[pl.ds(N_TOTAL + off, CHUNK)])
        return carry

    lax.fori_loop(0, STEPS, body, 0)


def kernel(x, W):
    idx = x.reshape(N_TOTAL).astype(jnp.int32)
    out = _gather_kernel(idx, W[:, 0], W[:, 1])
    return out.reshape(EMBED, BATCH, HIST).transpose(1, 2, 0)


# speedup vs baseline: 10.0728x; 1.0106x over previous
import functools

import jax
import jax.numpy as jnp
from jax import lax
from jax.experimental import pallas as pl
from jax.experimental.pallas import tpu as pltpu
from jax.experimental.pallas import tpu_sc as plsc

BATCH = 16384
HIST = 200
EMBED = 2
N_ROWS = 1_000_000
N_TOTAL = BATCH * HIST
NC, NS = 2, 16
NW = NC * NS
PER_W = N_TOTAL // NW
CHUNK = 12800
STEPS = PER_W // CHUNK

_mesh = plsc.VectorSubcoreMesh(core_axis_name="c", subcore_axis_name="s",
                               num_cores=NC, num_subcores=NS)


@functools.partial(
    pl.kernel,
    out_type=jax.ShapeDtypeStruct((EMBED * N_TOTAL,), jnp.float32),
    mesh=_mesh,
    scratch_types=[
        pltpu.VMEM((2, CHUNK), jnp.int32),
        pltpu.VMEM((2, CHUNK), jnp.float32),
        pltpu.VMEM((2, CHUNK), jnp.float32),
    ] + [pltpu.SemaphoreType.DMA] * 10,
    compiler_params=pltpu.CompilerParams(
        use_tc_tiling_on_sc=False, needs_layout_passes=False),
)
def _gather_kernel(idx_hbm, w0_hbm, w1_hbm, out_hbm, idx_v, e_v, o_v,
                   *sems):
    wid = lax.axis_index("s") * NC + lax.axis_index("c")
    base = wid * PER_W
    sem_i, sem_e, sem_o, sem_se, sem_so = (sems[0:2], sems[2:4], sems[4:6],
                                           sems[6:8], sems[8:10])

    def start_idx(g):
        off = base + g * CHUNK
        return pltpu.async_copy(idx_hbm.at[pl.ds(off, CHUNK)],
                                idx_v.at[g % 2], sem_i[g % 2])

    def start_gathers(g):
        b = g % 2
        return (pltpu.async_copy(w0_hbm.at[idx_v.at[b]], e_v.at[b], sem_e[b]),
                pltpu.async_copy(w1_hbm.at[idx_v.at[b]], o_v.at[b], sem_o[b]))

    def start_stores(g):
        b = g % 2
        off = base + g * CHUNK
        return (pltpu.async_copy(e_v.at[b], out_hbm.at[pl.ds(off, CHUNK)],
                                 sem_se[b]),
                pltpu.async_copy(o_v.at[b],
                                 out_hbm.at[pl.ds(N_TOTAL + off, CHUNK)],
                                 sem_so[b]))

    h_idx = [None] * STEPS
    h_g = [None] * STEPS
    h_s = [None] * STEPS
    h_idx[0] = start_idx(0)
    for g in range(STEPS):
        h_idx[g].wait()
        if g >= 2:
            h_s[g - 2][0].wait()
            h_s[g - 2][1].wait()
        h_g[g] = start_gathers(g)
        if g >= 1:
            h_g[g - 1][0].wait()
            h_g[g - 1][1].wait()
            h_s[g - 1] = start_stores(g - 1)
        if g + 1 < STEPS:
            h_idx[g + 1] = start_idx(g + 1)
    h_g[STEPS - 1][0].wait()
    h_g[STEPS - 1][1].wait()
    h_s[STEPS - 1] = start_stores(STEPS - 1)
    h_s[STEPS - 2][0].wait()
    h_s[STEPS - 2][1].wait()
    h_s[STEPS - 1][0].wait()
    h_s[STEPS - 1][1].wait()


def kernel(x, W):
    idx = x.reshape(N_TOTAL).astype(jnp.int32)
    out = _gather_kernel(idx, W[:, 0], W[:, 1])
    return out.reshape(EMBED, BATCH, HIST).transpose(1, 2, 0)
